# baseline (speedup 1.0000x reference)
import jax
import jax.numpy as jnp
import numpy as np
from jax.experimental import pallas as pl
from jax.experimental.pallas import tpu as pltpu
from jax.experimental.pallas import tpu_sc as plsc

_B = 2048
_W = 256


def _sc_gather(nodes2, idx, ep):
    feat = nodes2.shape[1]
    idx2 = idx.reshape(1, ep)
    mesh = plsc.VectorSubcoreMesh(core_axis_name="core", subcore_axis_name="subcore")

    @pl.kernel(out_type=jax.ShapeDtypeStruct((ep, feat), nodes2.dtype), mesh=mesh)
    def gather_kernel(x_hbm, i_hbm, o_hbm):
        def body(i_vmem, o_vmem):
            pltpu.sync_copy(x_hbm.at[i_vmem.at[0]], o_vmem)

        pltpu.emit_pipeline(
            body,
            grid=(ep // _W,),
            in_specs=[pl.BlockSpec((1, _W), index_map=lambda i: (0, i))],
            out_specs=[pl.BlockSpec((_W, feat), index_map=lambda i: (i, 0))],
            core_axis_name=("core", "subcore"),
            dimension_semantics=(pltpu.PARALLEL,),
        )(i_hbm, o_hbm)

    return gather_kernel(nodes2, idx2)


def _tc_messages(x128, par2, types2, kbig, sel, ep, in_f, rf, out_f):
    nb = ep // _B
    wide = 2 * in_f

    def body(x_ref, p_ref, t_ref, k_ref, s_ref, o_ref):
        xw = x_ref[...]
        pb = p_ref[...]
        tb = t_ref[...]
        col = jax.lax.broadcasted_iota(jnp.int32, (_B, wide), 1)
        xm = jnp.where((col // in_f) == pb, xw, 0.0).astype(jnp.bfloat16)
        y = jnp.dot(xm, k_ref[...], preferred_element_type=jnp.float32)
        rel = jax.lax.broadcasted_iota(jnp.int32, (_B, rf), 1) // out_f
        ym = jnp.where(rel == tb, y, 0.0)
        o_ref[...] = jnp.dot(ym, s_ref[...],
                             preferred_element_type=jnp.float32)

    return pl.pallas_call(
        body,
        grid=(nb,),
        in_specs=[
            pl.BlockSpec((_B, wide), lambda i: (i, 0)),
            pl.BlockSpec((_B, 1), lambda i: (i, 0)),
            pl.BlockSpec((_B, 1), lambda i: (i, 0)),
            pl.BlockSpec((wide, rf), lambda i: (0, 0)),
            pl.BlockSpec((rf, out_f), lambda i: (0, 0)),
        ],
        out_specs=pl.BlockSpec((_B, out_f), lambda i: (i, 0)),
        out_shape=jax.ShapeDtypeStruct((ep, out_f), jnp.float32),
        compiler_params=pltpu.CompilerParams(
            dimension_semantics=("parallel",)),
    )(x128, par2, types2, kbig, sel)


def kernel(nodes, senders, edge_types, kernels):
    e = senders.shape[0]
    num_rel, in_f, out_f = kernels.shape
    rf = num_rel * out_f
    nodes2 = nodes.reshape(nodes.shape[0] // 2, 2 * in_f)

    lcm = int(np.lcm(_B, _W))
    ep = ((e + lcm - 1) // lcm) * lcm
    pad = ep - e
    sp = jnp.pad(senders, (0, pad))
    tp = jnp.pad(edge_types, (0, pad))

    x128 = _sc_gather(nodes2, sp >> 1, ep)

    kflat = jnp.transpose(kernels, (1, 0, 2)).reshape(in_f, rf)
    kbig = jnp.concatenate([kflat, kflat], axis=0).astype(jnp.bfloat16)
    sel = jnp.tile(jnp.eye(out_f, dtype=jnp.float32), (num_rel, 1))

    out = _tc_messages(x128, (sp & 1).reshape(ep, 1),
                       tp.reshape(ep, 1), kbig, sel, ep, in_f, rf, out_f)
    return out[:e]
